# baseline (speedup 1.0000x reference)
import functools

import numpy as np
import jax
import jax.numpy as jnp
from jax import lax
from jax.experimental import pallas as pl
from jax.experimental.pallas import tpu as pltpu
from jax.experimental.pallas import tpu_sc as plsc

_VOCAB = 1000000
_D = 32
_B = 4096
_L = 50
_NC = 2
_NS = 16
_NW = _NC * _NS
_TOK = _B * _L
_PER_W = _TOK // _NW
_CH = 100
_NCH = _PER_W // _CH
_NB = 4
_ROUNDS = _NCH // _NB


def _pe_const() -> np.ndarray:
    pos = np.arange(_L, dtype=np.float32)[:, None]
    div = np.exp(np.arange(0, _D, 2, dtype=np.float32) * (-np.log(10000.0) / _D))
    pe = np.zeros((_L, _D), np.float32)
    pe[:, 0::2] = np.sin(pos * div)
    pe[:, 1::2] = np.cos(pos * div)
    return np.tile(pe, (_CH // _L, 1))


def kernel(indices, table):
    idx3 = jnp.reshape(indices.astype(jnp.int32), (_NW, _NCH, _CH))
    pe = jnp.asarray(_pe_const())

    mesh = plsc.VectorSubcoreMesh(core_axis_name="c", subcore_axis_name="s")

    @functools.partial(
        pl.kernel,
        mesh=mesh,
        compiler_params=pltpu.CompilerParams(use_tc_tiling_on_sc=False),
        out_type=jax.ShapeDtypeStruct((_TOK // _CH, _CH, _D), jnp.float32),
        scratch_types=(
            [pltpu.VMEM((_NCH, _CH), jnp.int32),
             pltpu.VMEM((_CH, _D), jnp.float32)]
            + [pltpu.VMEM((_CH, _D), jnp.float32) for _ in range(_NB)]
            + [pltpu.SemaphoreType.DMA for _ in range(2 * _NB)]
        ),
    )
    def run(table_hbm, idx_hbm, pe_hbm, out_hbm, idx_v, pe_v, *bufs_sems):
        bufs = bufs_sems[:_NB]
        gsem = bufs_sems[_NB:2 * _NB]
        ssem = bufs_sems[2 * _NB:]
        wid = lax.axis_index("s") * _NC + lax.axis_index("c")
        pltpu.sync_copy(idx_hbm.at[wid], idx_v)
        pltpu.sync_copy(pe_hbm, pe_v)
        base = wid * _NCH

        for b in range(_NB):
            pltpu.async_copy(table_hbm.at[idx_v.at[b]], bufs[b], gsem[b])

        def round_body(t, carry):
            for b in range(_NB):
                j = t * _NB + b
                buf = bufs[b]
                pltpu.make_async_copy(table_hbm.at[idx_v.at[j]], buf,
                                      gsem[b]).wait()
                for i in range(_CH):
                    for h in range(_D // 16):
                        sl = pl.ds(h * 16, 16)
                        buf[i, sl] = buf[i, sl] + pe_v[i, sl]
                pltpu.async_copy(buf, out_hbm.at[base + j], ssem[b])
                jn = lax.min(j + _NB, _NCH - 1)
                pltpu.make_async_copy(buf, out_hbm.at[base + j],
                                      ssem[b]).wait()
                pltpu.async_copy(table_hbm.at[idx_v.at[jn]], buf, gsem[b])
            return carry

        lax.fori_loop(0, _ROUNDS, round_body, 0)

        for b in range(_NB):
            pltpu.make_async_copy(table_hbm.at[idx_v.at[0]], bufs[b],
                                  gsem[b]).wait()

    return jnp.reshape(run(table, idx3, pe), (_B, _L, _D))


# speedup vs baseline: 1.1400x; 1.1400x over previous
import functools

import numpy as np
import jax
import jax.numpy as jnp
from jax import lax
from jax.experimental import pallas as pl
from jax.experimental.pallas import tpu as pltpu
from jax.experimental.pallas import tpu_sc as plsc

_VOCAB = 1000000
_D = 32
_B = 4096
_L = 50
_NC = 2
_NS = 16
_NW = _NC * _NS
_SEQ_W = _B // _NW
_NB = 8
_ROUNDS = _SEQ_W // _NB


def _pe_const() -> np.ndarray:
    pos = np.arange(_L, dtype=np.float32)[:, None]
    div = np.exp(np.arange(0, _D, 2, dtype=np.float32) * (-np.log(10000.0) / _D))
    pe = np.zeros((_L, _D), np.float32)
    pe[:, 0::2] = np.sin(pos * div)
    pe[:, 1::2] = np.cos(pos * div)
    return pe


def kernel(indices, table):
    idx = indices.astype(jnp.int32)
    pe = jnp.asarray(_pe_const())

    mesh = plsc.VectorSubcoreMesh(core_axis_name="c", subcore_axis_name="s")

    @functools.partial(
        pl.kernel,
        mesh=mesh,
        compiler_params=pltpu.CompilerParams(use_tc_tiling_on_sc=False),
        out_type=jax.ShapeDtypeStruct((_B, _L, _D), jnp.float32),
        scratch_types=(
            [pltpu.VMEM((_SEQ_W, _L), jnp.int32),
             pltpu.VMEM((_L, _D), jnp.float32)]
            + [pltpu.VMEM((_L, _D), jnp.float32) for _ in range(2 * _NB)]
            + [pltpu.SemaphoreType.DMA for _ in range(2 * _NB)]
        ),
    )
    def run(table_hbm, idx_hbm, pe_hbm, out_hbm, idx_v, pe_v, *bufs_sems):
        gbuf = bufs_sems[:_NB]
        sbuf = bufs_sems[_NB:2 * _NB]
        gsem = bufs_sems[2 * _NB:3 * _NB]
        ssem = bufs_sems[3 * _NB:]
        wid = lax.axis_index("s") * _NC + lax.axis_index("c")
        pltpu.sync_copy(idx_hbm.at[pl.ds(wid * _SEQ_W, _SEQ_W)], idx_v)
        pltpu.sync_copy(pe_hbm, pe_v)
        base = wid * _SEQ_W

        for b in range(_NB):
            pltpu.async_copy(table_hbm.at[idx_v.at[b]], gbuf[b], gsem[b])

        def round_body(t, carry):
            for b in range(_NB):
                j = t * _NB + b
                @pl.when(t > 0)
                def _wait_store():
                    pltpu.make_async_copy(sbuf[b], out_hbm.at[base + j],
                                          ssem[b]).wait()

                pltpu.make_async_copy(table_hbm.at[idx_v.at[j]], gbuf[b],
                                      gsem[b]).wait()
                for i in range(_L):
                    for h in range(_D // 16):
                        sl = pl.ds(h * 16, 16)
                        sbuf[b][i, sl] = gbuf[b][i, sl] + pe_v[i, sl]
                jn = lax.min(j + _NB, _SEQ_W - 1)
                pltpu.async_copy(table_hbm.at[idx_v.at[jn]], gbuf[b], gsem[b])
                pltpu.async_copy(sbuf[b], out_hbm.at[base + j], ssem[b])
            return carry

        lax.fori_loop(0, _ROUNDS, round_body, 0)

        for b in range(_NB):
            pltpu.make_async_copy(table_hbm.at[idx_v.at[0]], gbuf[b],
                                  gsem[b]).wait()
            pltpu.make_async_copy(sbuf[b], out_hbm.at[0], ssem[b]).wait()

    return run(table, idx, pe)
